# baseline (speedup 1.0000x reference)
import functools

import jax
import jax.numpy as jnp
from jax import lax
from jax.experimental import pallas as pl
from jax.experimental.pallas import tpu as pltpu
from jax.experimental.pallas import tpu_sc as plsc

_G = 16
_V = _G * _G * _G
_NC, _NS = 2, 16
_DG = 4
_CP = 8192


def _idx_body(x_ref, o_ref):
    x = x_ref[0]
    vi = jnp.clip((x * _G).astype(jnp.int32), 0, _G - 1)
    flat = vi[2] * (_G * _G) + vi[1] * _G + vi[0]
    j = lax.broadcasted_iota(jnp.int32, flat.shape, 0)
    ido = flat + (j & 15) * _V
    h = flat.shape[0] // 2
    o_ref[...] = ido[:h] | (ido[h:] << 16)


def _lin_body(f_ref, o_ref):
    r, n = f_ref.shape
    h = n // 2
    a = lax.bitcast_convert_type(
        f_ref[:, :h].astype(jnp.bfloat16), jnp.uint16).astype(jnp.int32)
    b = lax.bitcast_convert_type(
        f_ref[:, h:].astype(jnp.bfloat16), jnp.uint16).astype(jnp.int32)
    o_ref[...] = (a | (b << 16)).reshape(r * h)


def _div_body(s_ref, c_ref, o_ref):
    D = o_ref.shape[1]
    s = s_ref[...].reshape(D, _V)
    c = jnp.maximum(c_ref[...], 1.0)
    o_ref[0] = s / c[None, :]


def _build_sc(B, D, N):
    mesh = plsc.VectorSubcoreMesh(core_axis_name="c", subcore_axis_name="s")
    half = N // 2
    nchunks = half // _CP

    @functools.partial(
        pl.kernel,
        out_type=(
            jax.ShapeDtypeStruct((B * D * _V,), jnp.float32),
            jax.ShapeDtypeStruct((B * _V,), jnp.float32),
        ),
        mesh=mesh,
        scratch_types=[
            pltpu.VMEM((half,), jnp.int32),
            pltpu.VMEM((2, _CP), jnp.int32),
            pltpu.VMEM((16 * _V,), jnp.float32),
            pltpu.VMEM((_V,), jnp.float32),
            pltpu.VMEM((2, _V), jnp.float32),
            pltpu.SemaphoreType.DMA,
            pltpu.SemaphoreType.DMA,
        ],
        compiler_params=pltpu.CompilerParams(use_tc_tiling_on_sc=False,
                                             needs_layout_passes=False),
    )
    def sc(feat_hbm, idx_hbm, sums_hbm, cnts_hbm, idx_v, feat_v, bins,
           prev, outv, fsem, osem):
        cc = lax.axis_index("c")
        ss = lax.axis_index("s")
        wid = ss * _NC + cc
        b = wid // _DG
        dg = wid % _DG

        zvec = jnp.zeros((16,), jnp.float32)
        ones = jnp.full((16,), 1.0, jnp.float32)

        pltpu.sync_copy(idx_hbm.at[pl.ds(b * half, half)], idx_v)

        def zb(i, _):
            bins[pl.ds(i * 16, 16)] = zvec
            return 0

        lax.fori_loop(0, 16 * (_V // 16), zb, 0, unroll=8)

        def zp(i, _):
            prev[pl.ds(i * 16, 16)] = zvec
            return 0

        lax.fori_loop(0, _V // 16, zp, 0, unroll=8)

        def start_feat(fi):
            r = fi // nchunks
            ci = fi % nchunks
            d = dg * 16 + r
            rowbase = (b * D + d) * half
            buf = fi % 2
            pltpu.async_copy(
                feat_hbm.at[pl.ds(rowbase + ci * _CP, _CP)],
                feat_v.at[buf], fsem)

        def wait_feat(fi):
            buf = fi % 2
            pltpu.make_async_copy(
                feat_hbm.at[pl.ds(0, _CP)], feat_v.at[buf], fsem).wait()

        def reduce_row(dst_ref, base, obuf, r):
            @pl.when(r >= 2)
            def _():
                pltpu.make_async_copy(
                    sums_hbm.at[pl.ds(0, _V)], outv.at[0], osem).wait()

            def red(i, _):
                vs = [bins[pl.ds(l * _V + i * 16, 16)] for l in range(16)]
                while len(vs) > 1:
                    vs = [vs[k] + vs[k + 1] for k in range(0, len(vs), 2)]
                acc = vs[0]
                outv[obuf, pl.ds(i * 16, 16)] = acc - prev[pl.ds(i * 16, 16)]
                prev[pl.ds(i * 16, 16)] = acc
                return 0

            lax.fori_loop(0, _V // 16, red, 0, unroll=2)
            pltpu.async_copy(outv.at[obuf], dst_ref.at[pl.ds(base, _V)], osem)

        def scat(w, ve, vo):
            lo = w & 0xFFFF
            hi = lax.shift_right_logical(w, 16)
            plsc.addupdate_scatter(bins, [lo], ve)
            plsc.addupdate_scatter(bins, [hi], vo)

        start_feat(0)

        def chunk_step(fi, _):
            @pl.when(fi + 1 < 16 * nchunks)
            def _():
                start_feat(fi + 1)

            wait_feat(fi)
            buf = fi % 2
            ibase = (fi % nchunks) * _CP

            def loads(i):
                w = idx_v[pl.ds(ibase + i * 16, 16)]
                fp = feat_v[buf, pl.ds(i * 16, 16)]
                fe = plsc.bitcast(lax.shift_left(fp, 16), jnp.float32)
                fo = plsc.bitcast(fp & jnp.int32(-65536), jnp.float32)
                return w, fe, fo

            def ib(i, st):
                w, fe, fo = st
                nxt = loads(i + 1)
                scat(w, fe, fo)
                return nxt

            last = lax.fori_loop(0, _CP // 16 - 1, ib, loads(0), unroll=8)
            scat(*last)

            @pl.when(fi % nchunks == nchunks - 1)
            def _():
                r = fi // nchunks
                d = dg * 16 + r
                reduce_row(sums_hbm, (b * D + d) * _V, r % 2, r)

            return 0

        lax.fori_loop(0, 16 * nchunks, chunk_step, 0)

        pltpu.make_async_copy(sums_hbm.at[pl.ds(0, _V)], outv.at[0],
                              osem).wait()
        pltpu.make_async_copy(sums_hbm.at[pl.ds(0, _V)], outv.at[0],
                              osem).wait()

        @pl.when(dg == 0)
        def _counts():
            def ib(i, w):
                nxt = idx_v[pl.ds((i + 1) * 16, 16)]
                scat(w, ones, ones)
                return nxt

            last = lax.fori_loop(0, half // 16 - 1, ib,
                                 idx_v[pl.ds(0, 16)], unroll=4)
            scat(last, ones, ones)

            def red(i, _):
                vs = [bins[pl.ds(l * _V + i * 16, 16)] for l in range(16)]
                while len(vs) > 1:
                    vs = [vs[k] + vs[k + 1] for k in range(0, len(vs), 2)]
                outv[0, pl.ds(i * 16, 16)] = vs[0] - prev[pl.ds(i * 16, 16)]
                return 0

            lax.fori_loop(0, _V // 16, red, 0, unroll=2)
            pltpu.sync_copy(outv.at[0], cnts_hbm.at[pl.ds(b * _V, _V)])

    return sc


def kernel(features, xyz_normalized):
    B, D, N = features.shape
    xyz_t = jnp.transpose(xyz_normalized, (0, 2, 1))
    idx = pl.pallas_call(
        _idx_body,
        grid=(B,),
        in_specs=[pl.BlockSpec((1, 3, N), lambda b: (b, 0, 0))],
        out_specs=pl.BlockSpec((N // 2,), lambda b: (b,)),
        out_shape=jax.ShapeDtypeStruct((B * (N // 2),), jnp.int32),
    )(xyz_t)
    feat_lin = pl.pallas_call(
        _lin_body,
        grid=(B * D // 8,),
        in_specs=[pl.BlockSpec((8, N), lambda g: (g, 0))],
        out_specs=pl.BlockSpec((8 * (N // 2),), lambda g: (g,)),
        out_shape=jax.ShapeDtypeStruct((B * D * (N // 2),), jnp.int32),
    )(features.reshape(B * D, N))
    sums, cnts = _build_sc(B, D, N)(feat_lin, idx)
    out = pl.pallas_call(
        _div_body,
        grid=(B,),
        in_specs=[
            pl.BlockSpec((D * _V,), lambda b: (b,)),
            pl.BlockSpec((_V,), lambda b: (b,)),
        ],
        out_specs=pl.BlockSpec((1, D, _V), lambda b: (b, 0, 0)),
        out_shape=jax.ShapeDtypeStruct((B, D, _V), jnp.float32),
    )(sums, cnts)
    return out


# speedup vs baseline: 1.0177x; 1.0177x over previous
import functools

import jax
import jax.numpy as jnp
from jax import lax
from jax.experimental import pallas as pl
from jax.experimental.pallas import tpu as pltpu
from jax.experimental.pallas import tpu_sc as plsc

_G = 16
_V = _G * _G * _G
_NC, _NS = 2, 16
_DG = 4
_CP = 4096


def _idx_body(x_ref, o_ref):
    x = x_ref[0]
    vi = jnp.clip((x * _G).astype(jnp.int32), 0, _G - 1)
    flat = vi[2] * (_G * _G) + vi[1] * _G + vi[0]
    j = lax.broadcasted_iota(jnp.int32, flat.shape, 0)
    ido = flat + (j & 15) * _V
    h = flat.shape[0] // 2
    o_ref[...] = ido[:h] | (ido[h:] << 16)


def _lin_body(f_ref, o_ref):
    r, n = f_ref.shape
    h = n // 2
    a = lax.bitcast_convert_type(
        f_ref[:, :h].astype(jnp.bfloat16), jnp.uint16).astype(jnp.int32)
    b = lax.bitcast_convert_type(
        f_ref[:, h:].astype(jnp.bfloat16), jnp.uint16).astype(jnp.int32)
    o_ref[...] = (a | (b << 16)).reshape(r * h)


def _div_body(s_ref, c_ref, o_ref):
    D = o_ref.shape[1]
    s = s_ref[...].reshape(D, _V)
    c = jnp.maximum(c_ref[...], 1.0)
    o_ref[0] = s / c[None, :]


def _build_sc(B, D, N):
    mesh = plsc.VectorSubcoreMesh(core_axis_name="c", subcore_axis_name="s")
    half = N // 2
    nchunks = half // _CP

    @functools.partial(
        pl.kernel,
        out_type=(
            jax.ShapeDtypeStruct((B * D * _V,), jnp.float32),
            jax.ShapeDtypeStruct((B * _V,), jnp.float32),
        ),
        mesh=mesh,
        scratch_types=[
            pltpu.VMEM((half,), jnp.int32),
            pltpu.VMEM((2, _CP), jnp.int32),
            pltpu.VMEM((16 * _V,), jnp.float32),
            pltpu.VMEM((_V,), jnp.float32),
            pltpu.VMEM((2, _V), jnp.float32),
            pltpu.SemaphoreType.DMA,
            pltpu.SemaphoreType.DMA,
        ],
        compiler_params=pltpu.CompilerParams(use_tc_tiling_on_sc=False,
                                             needs_layout_passes=False),
    )
    def sc(feat_hbm, idx_hbm, sums_hbm, cnts_hbm, idx_v, feat_v, bins,
           prev, outv, fsem, osem):
        cc = lax.axis_index("c")
        ss = lax.axis_index("s")
        wid = ss * _NC + cc
        b = wid // _DG
        dg = wid % _DG

        zvec = jnp.zeros((16,), jnp.float32)
        ones = jnp.full((16,), 1.0, jnp.float32)

        pltpu.sync_copy(idx_hbm.at[pl.ds(b * half, half)], idx_v)

        def zb(i, _):
            bins[pl.ds(i * 16, 16)] = zvec
            return 0

        lax.fori_loop(0, 16 * (_V // 16), zb, 0, unroll=8)

        def zp(i, _):
            prev[pl.ds(i * 16, 16)] = zvec
            return 0

        lax.fori_loop(0, _V // 16, zp, 0, unroll=8)

        def start_feat(fi):
            r = fi // nchunks
            ci = fi % nchunks
            d = dg * 16 + r
            rowbase = (b * D + d) * half
            buf = fi % 2
            pltpu.async_copy(
                feat_hbm.at[pl.ds(rowbase + ci * _CP, _CP)],
                feat_v.at[buf], fsem)

        def wait_feat(fi):
            buf = fi % 2
            pltpu.make_async_copy(
                feat_hbm.at[pl.ds(0, _CP)], feat_v.at[buf], fsem).wait()

        def reduce_row(dst_ref, base, obuf, r):
            @pl.when(r >= 2)
            def _():
                pltpu.make_async_copy(
                    sums_hbm.at[pl.ds(0, _V)], outv.at[0], osem).wait()

            def red(i, _):
                vs = [bins[pl.ds(l * _V + i * 16, 16)] for l in range(16)]
                while len(vs) > 1:
                    vs = [vs[k] + vs[k + 1] for k in range(0, len(vs), 2)]
                acc = vs[0]
                outv[obuf, pl.ds(i * 16, 16)] = acc - prev[pl.ds(i * 16, 16)]
                prev[pl.ds(i * 16, 16)] = acc
                return 0

            lax.fori_loop(0, _V // 16, red, 0, unroll=2)
            pltpu.async_copy(outv.at[obuf], dst_ref.at[pl.ds(base, _V)], osem)

        def scat(w, ve, vo):
            lo = w & 0xFFFF
            hi = lax.shift_right_logical(w, 16)
            plsc.addupdate_scatter(bins, [lo], ve)
            plsc.addupdate_scatter(bins, [hi], vo)

        start_feat(0)

        def chunk_step(fi, _):
            @pl.when(fi + 1 < 16 * nchunks)
            def _():
                start_feat(fi + 1)

            wait_feat(fi)
            buf = fi % 2
            ibase = (fi % nchunks) * _CP

            def loads(i):
                w = idx_v[pl.ds(ibase + i * 16, 16)]
                fp = feat_v[buf, pl.ds(i * 16, 16)]
                fe = plsc.bitcast(lax.shift_left(fp, 16), jnp.float32)
                fo = plsc.bitcast(fp & jnp.int32(-65536), jnp.float32)
                return w, fe, fo

            def ib(i, st):
                w, fe, fo = st
                nxt = loads(i + 1)
                scat(w, fe, fo)
                return nxt

            last = lax.fori_loop(0, _CP // 16 - 1, ib, loads(0), unroll=4)
            scat(*last)

            @pl.when(fi % nchunks == nchunks - 1)
            def _():
                r = fi // nchunks
                d = dg * 16 + r
                reduce_row(sums_hbm, (b * D + d) * _V, r % 2, r)

            return 0

        lax.fori_loop(0, 16 * nchunks, chunk_step, 0)

        pltpu.make_async_copy(sums_hbm.at[pl.ds(0, _V)], outv.at[0],
                              osem).wait()
        pltpu.make_async_copy(sums_hbm.at[pl.ds(0, _V)], outv.at[0],
                              osem).wait()

        @pl.when(dg == 0)
        def _counts():
            def ib(i, w):
                nxt = idx_v[pl.ds((i + 1) * 16, 16)]
                scat(w, ones, ones)
                return nxt

            last = lax.fori_loop(0, half // 16 - 1, ib,
                                 idx_v[pl.ds(0, 16)], unroll=4)
            scat(last, ones, ones)

            def red(i, _):
                vs = [bins[pl.ds(l * _V + i * 16, 16)] for l in range(16)]
                while len(vs) > 1:
                    vs = [vs[k] + vs[k + 1] for k in range(0, len(vs), 2)]
                outv[0, pl.ds(i * 16, 16)] = vs[0] - prev[pl.ds(i * 16, 16)]
                return 0

            lax.fori_loop(0, _V // 16, red, 0, unroll=2)
            pltpu.sync_copy(outv.at[0], cnts_hbm.at[pl.ds(b * _V, _V)])

    return sc


def kernel(features, xyz_normalized):
    B, D, N = features.shape
    xyz_t = jnp.transpose(xyz_normalized, (0, 2, 1))
    idx = pl.pallas_call(
        _idx_body,
        grid=(B,),
        in_specs=[pl.BlockSpec((1, 3, N), lambda b: (b, 0, 0))],
        out_specs=pl.BlockSpec((N // 2,), lambda b: (b,)),
        out_shape=jax.ShapeDtypeStruct((B * (N // 2),), jnp.int32),
    )(xyz_t)
    feat_lin = pl.pallas_call(
        _lin_body,
        grid=(B * D // 8,),
        in_specs=[pl.BlockSpec((8, N), lambda g: (g, 0))],
        out_specs=pl.BlockSpec((8 * (N // 2),), lambda g: (g,)),
        out_shape=jax.ShapeDtypeStruct((B * D * (N // 2),), jnp.int32),
    )(features.reshape(B * D, N))
    sums, cnts = _build_sc(B, D, N)(feat_lin, idx)
    out = pl.pallas_call(
        _div_body,
        grid=(B,),
        in_specs=[
            pl.BlockSpec((D * _V,), lambda b: (b,)),
            pl.BlockSpec((_V,), lambda b: (b,)),
        ],
        out_specs=pl.BlockSpec((1, D, _V), lambda b: (b, 0, 0)),
        out_shape=jax.ShapeDtypeStruct((B, D, _V), jnp.float32),
    )(sums, cnts)
    return out
